# baseline (speedup 1.0000x reference)
import functools

import jax
import jax.numpy as jnp
from jax import lax
from jax.experimental import pallas as pl
from jax.experimental.pallas import tpu as pltpu
from jax.experimental.pallas import tpu_sc as plsc

_NUM_CORES = 2
_NUM_SUBCORES = 16
_NUM_WORKERS = _NUM_CORES * _NUM_SUBCORES
_CHUNK = 128
_WIN = 2048
_NWIN = 50
_BCAP = 768
_POS_BITS = 20
_POS_MASK = (1 << _POS_BITS) - 1


@jax.jit
def _embed_lookup(idx2d, table_pad):
    n_rows, chunk = idx2d.shape
    vpad, d = table_pad.shape
    b = n_rows * chunk
    chunks_per_w = n_rows // _NUM_WORKERS
    per_w = chunks_per_w * chunk
    trash = b

    mesh = plsc.VectorSubcoreMesh(core_axis_name="c", subcore_axis_name="s")

    @functools.partial(
        pl.kernel,
        out_type=jax.ShapeDtypeStruct((b + 128, d), jnp.float32),
        mesh=mesh,
        compiler_params=pltpu.CompilerParams(needs_layout_passes=False),
        scratch_types=[
            pltpu.VMEM((chunks_per_w, chunk), jnp.int32),
            pltpu.VMEM((_NWIN * _BCAP,), jnp.int32),
            pltpu.VMEM((64,), jnp.int32),
            pltpu.VMEM((2, chunk), jnp.int32),
            pltpu.VMEM((2, chunk), jnp.int32),
            pltpu.VMEM((2, chunk, d), jnp.float32),
            pltpu.VMEM_SHARED((2, _WIN, d), jnp.float32),
            pltpu.SemaphoreType.DMA((2,)),
            pltpu.SemaphoreType.DMA((2,)),
            pltpu.SemaphoreType.DMA((2,)),
        ],
    )
    def gather_kernel(idx_hbm, table_hbm, out_hbm, idx_v, packed_v, cnt_v,
                      widx_v, pos_v, rows_v, win_sh, stsem, gsem, wsem):
        cid = lax.axis_index("c")
        sid = lax.axis_index("s")
        wid = sid * _NUM_CORES + cid
        base = wid * chunks_per_w
        iota = lax.iota(jnp.int32, 16)
        zeros = jnp.zeros((16,), jnp.int32)

        def stage_window(w):
            pltpu.async_copy(
                table_hbm.at[pl.ds(w * _WIN + sid * (_WIN // 16), _WIN // 16)],
                win_sh.at[w % 2].at[pl.ds(sid * (_WIN // 16), _WIN // 16)],
                stsem.at[w % 2],
            )

        pltpu.sync_copy(idx_hbm.at[pl.ds(base, chunks_per_w)], idx_v)
        stage_window(0)
        stage_window(1)

        for g in range(4):
            cnt_v[pl.ds(g * 16, 16)] = zeros
        def init_body(i, carry):
            padval = trash + ((i * 16 + iota) & 127)
            packed_v[pl.ds(i * 16, 16)] = padval
            return carry

        lax.fori_loop(0, _NWIN * _BCAP // 16, init_body, 0)

        def bin_body(j, carry):
            for q in range(chunk // 16):
                idxv = idx_v[j, pl.ds(q * 16, 16)]
                w = lax.shift_right_logical(idxv, 11)
                widx = idxv & (_WIN - 1)
                pos = base * chunk + j * chunk + q * 16 + iota
                packed = lax.shift_left(widx, _POS_BITS) | pos
                rank = jnp.zeros((16,), jnp.int32)
                dnums = lax.GatherDimensionNumbers(
                    offset_dims=(), collapsed_slice_dims=(0,),
                    start_index_map=(0,),
                )
                for k in range(1, 16):
                    rolled = lax.gather(
                        w, ((iota - k) & 15)[:, None], dnums, (1,),
                        mode=lax.GatherScatterMode.PROMISE_IN_BOUNDS,
                    )
                    rank = rank + jnp.where(
                        (iota >= k) & (w == rolled), 1, 0
                    ).astype(jnp.int32)
                bs = plsc.load_gather(cnt_v, [w])
                plsc.addupdate_scatter(cnt_v, [w], jnp.ones((16,), jnp.int32))
                slot = w * _BCAP + bs + rank
                plsc.store_scatter(packed_v, [slot], packed)
            return carry

        lax.fori_loop(0, chunks_per_w, bin_body, 0)

        def unpack(pb, k, slot):
            for q in range(chunk // 16):
                pk = packed_v[pl.ds(pb + k * chunk + q * 16, 16)]
                widx_v[slot, pl.ds(q * 16, 16)] = (
                    lax.shift_right_logical(pk, _POS_BITS) & (_WIN - 1)
                )
                pos_v[slot, pl.ds(q * 16, 16)] = pk & _POS_MASK

        def issue_gather(parity, slot):
            pltpu.async_copy(
                win_sh.at[parity].at[widx_v.at[slot]],
                rows_v.at[slot],
                gsem.at[slot],
            )

        def wait_gather(parity, slot):
            pltpu.make_async_copy(
                win_sh.at[parity].at[widx_v.at[slot]],
                rows_v.at[slot],
                gsem.at[slot],
            ).wait()

        def issue_scatter(slot):
            pltpu.async_copy(
                rows_v.at[slot], out_hbm.at[pos_v.at[slot]], wsem.at[slot]
            )

        def wait_scatter(slot):
            pltpu.make_async_copy(
                rows_v.at[slot], out_hbm.at[pos_v.at[slot]], wsem.at[slot]
            ).wait()

        cnt_grps = [cnt_v[pl.ds(g * 16, 16)] for g in range(4)]

        def window_body(w, parity):
            wdiv = w // 16
            cvec = jnp.where(
                wdiv == 0,
                cnt_grps[0],
                jnp.where(
                    wdiv == 1,
                    cnt_grps[1],
                    jnp.where(wdiv == 2, cnt_grps[2], cnt_grps[3]),
                ),
            )
            c_w = jnp.sum(jnp.where(iota == (w % 16), cvec, 0))
            n2 = (c_w + 2 * chunk - 1) // (2 * chunk)
            pb = w * _BCAP

            pltpu.make_async_copy(
                table_hbm.at[pl.ds(0, _WIN // 16)],
                win_sh.at[parity].at[pl.ds(0, _WIN // 16)],
                stsem.at[parity],
            ).wait()
            plsc.subcore_barrier()

            def pair_body(i2, carry):
                @pl.when(i2 > 0)
                def _():
                    wait_scatter(0)
                    wait_scatter(1)

                unpack(pb, 2 * i2, 0)
                issue_gather(parity, 0)
                unpack(pb, 2 * i2 + 1, 1)
                issue_gather(parity, 1)
                wait_gather(parity, 0)
                issue_scatter(0)
                wait_gather(parity, 1)
                issue_scatter(1)
                return carry

            lax.fori_loop(0, n2, pair_body, 0)

            @pl.when(n2 > 0)
            def _():
                wait_scatter(0)
                wait_scatter(1)

            plsc.subcore_barrier()

            @pl.when(w + 2 < _NWIN)
            def _():
                stage_window_dyn(w + 2, parity)

        def stage_window_dyn(w, parity):
            pltpu.async_copy(
                table_hbm.at[pl.ds(w * _WIN + sid * (_WIN // 16), _WIN // 16)],
                win_sh.at[parity].at[pl.ds(sid * (_WIN // 16), _WIN // 16)],
                stsem.at[parity],
            )

        def wpair_body(wp, carry):
            window_body(2 * wp, 0)
            window_body(2 * wp + 1, 1)
            return carry

        lax.fori_loop(0, _NWIN // 2, wpair_body, 0)

    return gather_kernel(idx2d, table_pad)


def kernel(X, table):
    b0, s = X.shape
    v, d = table.shape
    b = b0 * s
    idx2d = X.reshape(b // _CHUNK, _CHUNK).astype(jnp.int32)
    table_pad = jnp.pad(table, ((0, _NWIN * _WIN - v), (0, 0)))
    out = _embed_lookup(idx2d, table_pad)
    return out[:b].reshape(b0, s, d)


# speedup vs baseline: 2.3902x; 2.3902x over previous
import functools

import jax
import jax.numpy as jnp
from jax import lax
from jax.experimental import pallas as pl
from jax.experimental.pallas import tpu as pltpu
from jax.experimental.pallas import tpu_sc as plsc

_NUM_CORES = 2
_NUM_SUBCORES = 16
_NUM_WORKERS = _NUM_CORES * _NUM_SUBCORES
_CHUNK = 128
_NBUF = 5


@jax.jit
def _embed_lookup(idx2d, table):
    n_rows, chunk = idx2d.shape
    v, d = table.shape
    b = n_rows * chunk
    chunks_per_w = n_rows // _NUM_WORKERS
    ngroups = chunks_per_w // _NBUF

    mesh = plsc.VectorSubcoreMesh(core_axis_name="c", subcore_axis_name="s")

    @functools.partial(
        pl.kernel,
        out_type=jax.ShapeDtypeStruct((b, d), jnp.float32),
        mesh=mesh,
        scratch_types=[
            pltpu.VMEM((chunks_per_w, chunk), jnp.int32),
            pltpu.VMEM((_NBUF, chunk, d), jnp.float32),
            pltpu.SemaphoreType.DMA((_NBUF,)),
            pltpu.SemaphoreType.DMA((_NBUF,)),
        ],
    )
    def gather_kernel(idx_hbm, table_hbm, out_hbm, idx_v, rows_v, gsem, wsem):
        wid = lax.axis_index("s") * _NUM_CORES + lax.axis_index("c")
        base = wid * chunks_per_w
        pltpu.sync_copy(idx_hbm.at[pl.ds(base, chunks_per_w)], idx_v)

        def issue_gather(j, slot):
            pltpu.async_copy(
                table_hbm.at[idx_v.at[j]], rows_v.at[slot], gsem.at[slot]
            )

        def wait_gather(slot):
            pltpu.make_async_copy(
                table_hbm.at[idx_v.at[0]], rows_v.at[slot], gsem.at[slot]
            ).wait()

        def issue_write(j, slot):
            pltpu.async_copy(
                rows_v.at[slot],
                out_hbm.at[pl.ds((base + j) * chunk, chunk)],
                wsem.at[slot],
            )

        def wait_write(slot):
            pltpu.make_async_copy(
                rows_v.at[slot],
                out_hbm.at[pl.ds(base * chunk, chunk)],
                wsem.at[slot],
            ).wait()

        issue_gather(0, 0)
        issue_gather(1, 1)

        for s in range(_NBUF):
            wait_gather(s)
            issue_write(s, s)
            nxt = s + 2
            if nxt >= _NBUF:
                wait_write(nxt % _NBUF)
            issue_gather(nxt, nxt % _NBUF)

        def body(g, carry):
            for s in range(_NBUF):
                j = g * _NBUF + s
                wait_gather(s)
                issue_write(j, s)
                wait_write((s + 2) % _NBUF)
                issue_gather(j + 2, (s + 2) % _NBUF)
            return carry

        lax.fori_loop(1, ngroups - 1, body, 0)

        last = (ngroups - 1) * _NBUF
        for s in range(_NBUF):
            j = last + s
            wait_gather(s)
            issue_write(j, s)
            if j + 2 < chunks_per_w:
                wait_write((s + 2) % _NBUF)
                issue_gather(j + 2, (s + 2) % _NBUF)
        for s in range(_NBUF):
            wait_write(s)

    return gather_kernel(idx2d, table)


def kernel(X, table):
    b0, s = X.shape
    v, d = table.shape
    b = b0 * s
    idx2d = X.reshape(b // _CHUNK, _CHUNK).astype(jnp.int32)
    out = _embed_lookup(idx2d, table)
    return out.reshape(b0, s, d)
